# speedup vs baseline: 10.6264x; 10.6264x over previous
import numpy as np
import jax
import jax.numpy as jnp
from jax import lax
from jax.experimental import pallas as pl
from jax.experimental.pallas import tpu as pltpu

_N = 128
_K = 512
_BS = 8192
_BLK = 512
_H = np.float32(1.0 / 127.0)
_C1 = np.float32(127.0 / 6.0)
_C2 = np.float32(1.0 / 762.0)


def _build_P() -> np.ndarray:
    n = _N
    h = 1.0 / (n - 1)
    idx = np.arange(n - 2)
    S = np.zeros((n - 2, n))
    S[idx, idx] = 6.0 / h
    S[idx, idx + 1] = -12.0 / h
    S[idx, idx + 2] = 6.0 / h
    A = (np.diag(4.0 * h * np.ones(n - 2))
         + np.diag(h * np.ones(n - 3), 1)
         + np.diag(h * np.ones(n - 3), -1))
    P = np.zeros((n, n))
    P[1:-1] = np.linalg.solve(A, S)
    return P.astype(np.float32)


_P_CONST = _build_P()


def _mtable_body(p_ref, x_ref, m_ref):
    m_ref[:] = jnp.dot(p_ref[:], x_ref[:],
                       preferred_element_type=jnp.float32,
                       precision=lax.Precision.HIGHEST)


def _eval_body(t_ref, yg_ref, sg_ref, mmu_ref, msg_ref, mu_ref, sig_ref):
    t = t_ref[:]
    tf = t * np.float32(127.0)
    fidx = jnp.clip(jnp.floor(tf), 0.0, 126.0)
    i = fidx.astype(jnp.int32)
    x0 = fidx * _H
    x1 = (fidx + np.float32(1.0)) * _H
    dx0 = t - x0
    dx1 = x1 - t
    a = dx1 * np.float32(127.0)
    b = dx0 * np.float32(127.0)
    c = dx1 * (_C1 * dx1 * dx1 - _C2)
    d = dx0 * (_C1 * dx0 * dx0 - _C2)

    iota = lax.broadcasted_iota(jnp.int32, (_BLK, _N), 1)
    e0 = iota == i[:, None]
    e1 = iota == (i + 1)[:, None]
    zero = jnp.zeros((), jnp.float32)
    hmat = jnp.where(e0, a[:, None], zero) + jnp.where(e1, b[:, None], zero)
    gmat = jnp.where(e0, c[:, None], zero) + jnp.where(e1, d[:, None], zero)

    dot = lambda x, y: jnp.dot(x, y, preferred_element_type=jnp.float32,
                               precision=lax.Precision.HIGHEST)
    mu_ref[:] = dot(hmat, yg_ref[:]) + dot(gmat, mmu_ref[:])
    sig_ref[:] = jnp.exp(dot(hmat, sg_ref[:]) + dot(gmat, msg_ref[:]))


def kernel(t, mu_params, sigma_params, w_logits):
    ones_row = jnp.ones((1, _K), jnp.float32)
    y_grid = jnp.concatenate([-ones_row, mu_params, ones_row], axis=0)
    s_grid = jnp.concatenate([0.0 * ones_row, sigma_params, 0.0 * ones_row], axis=0)

    p_const = jnp.asarray(_P_CONST)
    x_both = jnp.concatenate([y_grid, s_grid], axis=1)
    m_both = pl.pallas_call(
        _mtable_body,
        out_shape=jax.ShapeDtypeStruct((_N, 2 * _K), jnp.float32),
    )(p_const, x_both)
    m_mu = m_both[:, :_K]
    m_sg = m_both[:, _K:]

    nblk = _BS // _BLK
    full = pl.BlockSpec((_N, _K), lambda b: (0, 0))
    mu, sig = pl.pallas_call(
        _eval_body,
        grid=(nblk,),
        in_specs=[
            pl.BlockSpec((_BLK,), lambda b: (b,)),
            full, full, full, full,
        ],
        out_specs=[
            pl.BlockSpec((_BLK, _K), lambda b: (b, 0)),
            pl.BlockSpec((_BLK, _K), lambda b: (b, 0)),
        ],
        out_shape=[
            jax.ShapeDtypeStruct((_BS, _K), jnp.float32),
            jax.ShapeDtypeStruct((_BS, _K), jnp.float32),
        ],
    )(t, y_grid, s_grid, m_mu, m_sg)

    return (mu.reshape(_BS, 8, 64), sig.reshape(_BS, 8, 64), w_logits)
